# baseline (speedup 1.0000x reference)
import functools

import jax
import jax.numpy as jnp
from jax import lax
from jax.experimental import pallas as pl
from jax.experimental.pallas import tpu as pltpu
from jax.experimental.pallas import tpu_sc as plsc

_VOCAB = 1000000
_EMBED = 64
_BATCH = 4096
_SEQ = 200
_HALF = _EMBED // 2

_NC = 2
_NS = 16
_NW = _NC * _NS

_TOTAL = _BATCH * _SEQ
_PER_W = _TOTAL // _NW
_CHUNK = 512
_IDXR = _CHUNK // 128
_NCHUNK = _PER_W // _CHUNK


def _sc_body(x2d, sincos, table, out, idx_v, rows_v, sc_v, sem):
    wid = lax.axis_index("s") * _NC + lax.axis_index("c")
    pltpu.sync_copy(sincos, sc_v)

    def chunk_body(g, carry):
        irow0 = wid * (_PER_W // 128) + g * _IDXR
        base = irow0 * 128
        pltpu.sync_copy(x2d.at[pl.ds(irow0, _IDXR)], idx_v)
        copies = []
        for j in range(_IDXR):
            copies.append(
                pltpu.async_copy(
                    table.at[idx_v.at[j]],
                    rows_v.at[pl.ds(j * 128, 128)],
                    sem,
                )
            )
        for c in copies:
            c.wait()

        t0 = lax.rem(g * _CHUNK, _SEQ)

        def row_body(r, t):
            e0 = rows_v[r, 0:16]
            e1 = rows_v[r, 16:32]
            o0 = rows_v[r, 32:48]
            o1 = rows_v[r, 48:64]
            c0 = sc_v[t, 0:16]
            c1 = sc_v[t, 16:32]
            s0 = sc_v[t, 32:48]
            s1 = sc_v[t, 48:64]
            rows_v[r, 0:16] = e0 * c0 - o0 * s0
            rows_v[r, 16:32] = e1 * c1 - o1 * s1
            rows_v[r, 32:48] = e0 * s0 + o0 * c0
            rows_v[r, 48:64] = e1 * s1 + o1 * c1
            t1 = t + 1
            return jnp.where(t1 >= _SEQ, 0, t1)

        lax.fori_loop(0, _CHUNK, row_body, t0)
        pltpu.sync_copy(rows_v, out.at[pl.ds(base, _CHUNK)])
        return carry

    lax.fori_loop(0, _NCHUNK, chunk_body, 0)


@jax.jit
def _sc_call(x2d, sincos, table):
    mesh = plsc.VectorSubcoreMesh(core_axis_name="c", subcore_axis_name="s")
    f = pl.kernel(
        _sc_body,
        mesh=mesh,
        compiler_params=pltpu.CompilerParams(use_tc_tiling_on_sc=False),
        out_type=jax.ShapeDtypeStruct((_TOTAL, _EMBED), jnp.float32),
        scratch_types=[
            pltpu.VMEM((_IDXR, 128), jnp.int32),
            pltpu.VMEM((_CHUNK, _EMBED), jnp.float32),
            pltpu.VMEM((_SEQ, _EMBED), jnp.float32),
            pltpu.SemaphoreType.DMA,
        ],
    )
    return f(x2d, sincos, table)


def kernel(x, table):
    if x.ndim == 1:
        x = x[None, :]
    x2d = x.astype(jnp.int32).reshape(_TOTAL // 128, 128)
    freqs = 1.0 / (10000.0 ** (jnp.arange(_HALF, dtype=jnp.float32) / _EMBED))
    ang = jnp.arange(_SEQ, dtype=jnp.float32)[:, None] * freqs[None, :]
    sincos = jnp.concatenate([jnp.cos(ang), jnp.sin(ang)], axis=-1)
    out = _sc_call(x2d, sincos, table)
    return out.reshape(_BATCH, _SEQ, _EMBED)


# speedup vs baseline: 1.2050x; 1.2050x over previous
import jax
import jax.numpy as jnp
from jax import lax
from jax.experimental import pallas as pl
from jax.experimental.pallas import tpu as pltpu
from jax.experimental.pallas import tpu_sc as plsc

_VOCAB = 1000000
_EMBED = 64
_BATCH = 4096
_SEQ = 200
_HALF = _EMBED // 2

_NC = 2
_NS = 16
_NW = _NC * _NS

_TOTAL = _BATCH * _SEQ
_BPW = _BATCH // _NW
_NBUF = 5
_KMAX = _SEQ // _NBUF


def _sc_body(xtw, oidx, sincos, table, out,
             idx_v, odx_v, sc_v,
             r0, r1, r2, r3, r4,
             si0, si1, si2, si3, si4,
             so0, so1, so2, so3, so4):
    rows = [r0, r1, r2, r3, r4]
    sin_ = [si0, si1, si2, si3, si4]
    sout = [so0, so1, so2, so3, so4]
    wid = lax.axis_index("s") * _NC + lax.axis_index("c")

    pltpu.sync_copy(xtw.at[wid], idx_v)
    pltpu.sync_copy(oidx.at[wid], odx_v)
    pltpu.sync_copy(sincos, sc_v)

    def block(k, carry):
        g0 = _NBUF * k
        hin = [
            pltpu.async_copy(table.at[idx_v.at[g0 + b]], rows[b], sin_[b])
            for b in range(_NBUF)
        ]
        hout = []
        for b in range(_NBUF):
            g = g0 + b
            hin[b].wait()

            rb = rows[b]
            c0 = sc_v[g, 0:16]
            c1 = sc_v[g, 16:32]
            s0 = sc_v[g, 32:48]
            s1 = sc_v[g, 48:64]

            @plsc.parallel_loop(0, _BPW, unroll=4)
            def _(j):
                e0 = rb[j, 0:16]
                e1 = rb[j, 16:32]
                o0 = rb[j, 32:48]
                o1 = rb[j, 48:64]
                rb[j, 0:16] = e0 * c0 - o0 * s0
                rb[j, 16:32] = e1 * c1 - o1 * s1
                rb[j, 32:48] = e0 * s0 + o0 * c0
                rb[j, 48:64] = e1 * s1 + o1 * c1

            hout.append(pltpu.async_copy(rb, out.at[odx_v.at[g]], sout[b]))
        for h in hout:
            h.wait()
        return carry

    lax.fori_loop(0, _KMAX, block, 0)


@jax.jit
def _sc_call(xtw, oidx, sincos, table):
    mesh = plsc.VectorSubcoreMesh(core_axis_name="c", subcore_axis_name="s")
    f = pl.kernel(
        _sc_body,
        mesh=mesh,
        compiler_params=pltpu.CompilerParams(use_tc_tiling_on_sc=False),
        out_type=jax.ShapeDtypeStruct((_TOTAL, _EMBED), jnp.float32),
        scratch_types=[
            pltpu.VMEM((_SEQ, _BPW), jnp.int32),
            pltpu.VMEM((_SEQ, _BPW), jnp.int32),
            pltpu.VMEM((_SEQ, _EMBED), jnp.float32),
        ] + [pltpu.VMEM((_BPW, _EMBED), jnp.float32)] * _NBUF
          + [pltpu.SemaphoreType.DMA] * (2 * _NBUF),
    )
    return f(xtw, oidx, sincos, table)


def kernel(x, table):
    if x.ndim == 1:
        x = x[None, :]
    xtw = x.astype(jnp.int32).reshape(_NW, _BPW, _SEQ).transpose(0, 2, 1)
    bgrid = (jnp.arange(_NW, dtype=jnp.int32)[:, None, None] * _BPW
             + jnp.arange(_BPW, dtype=jnp.int32)[None, None, :])
    oidx = bgrid * _SEQ + jnp.arange(_SEQ, dtype=jnp.int32)[None, :, None]
    freqs = 1.0 / (10000.0 ** (jnp.arange(_HALF, dtype=jnp.float32) / _EMBED))
    ang = jnp.arange(_SEQ, dtype=jnp.float32)[:, None] * freqs[None, :]
    sincos = jnp.concatenate([jnp.cos(ang), jnp.sin(ang)], axis=-1)
    out = _sc_call(xtw, oidx, sincos, table)
    return out.reshape(_BATCH, _SEQ, _EMBED)
